# baseline (speedup 1.0000x reference)
import jax
import jax.numpy as jnp
from jax.experimental import pallas as pl
from jax.experimental.pallas import tpu as pltpu


def _conv_stats_kernel(x_ref, w_ref, b_ref, y_ref, st_ref, x3_ref):
    Cin = x_ref.shape[1]
    HW = x_ref.shape[2]
    HPW = x3_ref.shape[1]
    W = (HPW - HW) // 2

    x = x_ref[0]

    lane = jax.lax.broadcasted_iota(jnp.int32, (Cin, HW), 1) % W
    left = jnp.concatenate([x[:, :1], x[:, :-1]], axis=1)
    right = jnp.concatenate([x[:, 1:], x[:, -1:]], axis=1)
    xl = jnp.where(lane == 0, right, left)
    xr = jnp.where(lane == W - 1, left, right)

    for i, vb in enumerate((xl, x, xr)):
        r0 = i * Cin
        x3_ref[r0:r0 + Cin, W:W + HW] = vb
        x3_ref[r0:r0 + Cin, 0:W] = vb[:, W:2 * W]
        x3_ref[r0:r0 + Cin, W + HW:HPW] = vb[:, HW - 2 * W:HW - W]

    acc = None
    for dy in range(3):
        contrib = jnp.dot(w_ref[dy], x3_ref[:, dy * W:dy * W + HW],
                          preferred_element_type=jnp.float32)
        acc = contrib if acc is None else acc + contrib
    acc = acc + b_ref[...]

    y_ref[0] = acc.astype(jnp.bfloat16)

    s = jnp.sum(acc, axis=1, keepdims=True)
    ss = jnp.sum(acc * acc, axis=1, keepdims=True)
    st_ref[0] = jnp.concatenate([s, ss], axis=1)


def _bn_relu_kernel(y_ref, sc_ref, sh_ref, o_ref):
    z = y_ref[0].astype(jnp.float32) * sc_ref[...] + sh_ref[...]
    o_ref[0] = jnp.maximum(z, 0.0).astype(jnp.bfloat16)


def kernel(x_nchw, weight, bias, gamma, beta):
    eps = 1e-5
    x = x_nchw.astype(jnp.float32)
    N, Cin, H, W = x.shape
    Cout = weight.shape[0]
    HW = H * W
    HPW = (H + 2) * W

    xf = x.reshape(N, Cin, HW).astype(jnp.bfloat16)
    w_r = (jnp.transpose(weight.astype(jnp.float32), (2, 0, 3, 1))
           .reshape(3, Cout, 3 * Cin).astype(jnp.bfloat16))
    b2 = bias.astype(jnp.float32).reshape(Cout, 1)

    y, st = pl.pallas_call(
        _conv_stats_kernel,
        out_shape=(jax.ShapeDtypeStruct((N, Cout, HW), jnp.bfloat16),
                   jax.ShapeDtypeStruct((N, Cout, 2), jnp.float32)),
        name="conv3x3_stats",
        grid=(N,),
        in_specs=[pl.BlockSpec((1, Cin, HW), lambda g: (g, 0, 0)),
                  pl.BlockSpec((3, Cout, 3 * Cin), lambda g: (0, 0, 0)),
                  pl.BlockSpec((Cout, 1), lambda g: (0, 0))],
        out_specs=(pl.BlockSpec((1, Cout, HW), lambda g: (g, 0, 0)),
                   pl.BlockSpec((1, Cout, 2), lambda g: (g, 0, 0))),
        scratch_shapes=[pltpu.VMEM((3 * Cin, HPW), jnp.bfloat16)],
        compiler_params=pltpu.CompilerParams(
            dimension_semantics=("parallel",),
            vmem_limit_bytes=64 * 1024 * 1024),
    )(xf, w_r, b2)

    cnt = float(N * HW)
    s = jnp.sum(st[:, :, 0], axis=0)
    ss = jnp.sum(st[:, :, 1], axis=0)
    mean = s / cnt
    var = jnp.maximum(ss / cnt - mean * mean, 0.0)
    scale = gamma.astype(jnp.float32) / jnp.sqrt(var + eps)
    shift = beta.astype(jnp.float32) - mean * scale
    scale2 = scale.reshape(Cout, 1)
    shift2 = shift.reshape(Cout, 1)

    NL = 2
    TL = HW // NL
    out = pl.pallas_call(
        _bn_relu_kernel,
        out_shape=jax.ShapeDtypeStruct((N, Cout, HW), jnp.bfloat16),
        name="bn_relu",
        grid=(N, NL),
        in_specs=[pl.BlockSpec((1, Cout, TL), lambda n, l: (n, 0, l)),
                  pl.BlockSpec((Cout, 1), lambda n, l: (0, 0)),
                  pl.BlockSpec((Cout, 1), lambda n, l: (0, 0))],
        out_specs=pl.BlockSpec((1, Cout, TL), lambda n, l: (n, 0, l)),
        compiler_params=pltpu.CompilerParams(
            dimension_semantics=("parallel", "parallel"),
            vmem_limit_bytes=64 * 1024 * 1024),
    )(y, scale2, shift2)

    return out.astype(jnp.float32).reshape(N, Cout, H, W)


# speedup vs baseline: 2.4650x; 2.4650x over previous
import jax
import jax.numpy as jnp
from jax.experimental import pallas as pl
from jax.experimental.pallas import tpu as pltpu


def _conv_stats_kernel(x_ref, w_ref, b_ref, y_ref, st_ref, x3_ref):
    Cin = x_ref.shape[1]
    HW = x_ref.shape[2] * x_ref.shape[3] * x_ref.shape[4]
    HPW = x3_ref.shape[1]
    W = (HPW - HW) // 2

    x = x_ref[0].astype(jnp.bfloat16).reshape(Cin, HW)

    lane = jax.lax.broadcasted_iota(jnp.int32, (Cin, HW), 1) % W
    left = jnp.concatenate([x[:, :1], x[:, :-1]], axis=1)
    right = jnp.concatenate([x[:, 1:], x[:, -1:]], axis=1)
    xl = jnp.where(lane == 0, right, left)
    xr = jnp.where(lane == W - 1, left, right)

    for i, vb in enumerate((xl, x, xr)):
        r0 = i * Cin
        x3_ref[r0:r0 + Cin, W:W + HW] = vb
        x3_ref[r0:r0 + Cin, 0:W] = vb[:, W:2 * W]
        x3_ref[r0:r0 + Cin, W + HW:HPW] = vb[:, HW - 2 * W:HW - W]

    acc = None
    for dy in range(3):
        contrib = jnp.dot(w_ref[dy], x3_ref[:, dy * W:dy * W + HW],
                          preferred_element_type=jnp.float32)
        acc = contrib if acc is None else acc + contrib
    acc = acc + b_ref[...]

    y_ref[0] = acc.astype(jnp.bfloat16)

    s = jnp.sum(acc, axis=1, keepdims=True)
    ss = jnp.sum(acc * acc, axis=1, keepdims=True)
    st_ref[0] = jnp.concatenate([s, ss], axis=1)


def _bn_relu_kernel(y_ref, sc_ref, sh_ref, o_ref):
    C, H8, S, W = o_ref.shape[1:]
    yv = y_ref[0].reshape(C, H8, S, W)
    sc = sc_ref[...].reshape(C, 1, 1, 1)
    sh = sh_ref[...].reshape(C, 1, 1, 1)
    z = yv.astype(jnp.float32) * sc + sh
    o_ref[0] = jnp.maximum(z, 0.0)


def kernel(x_nchw, weight, bias, gamma, beta):
    eps = 1e-5
    x = x_nchw.astype(jnp.float32)
    N, Cin, H, W = x.shape
    Cout = weight.shape[0]
    HW = H * W
    HPW = (H + 2) * W

    H8 = H // 8
    xf = x.reshape(N, Cin, H8, 8, W)
    w_r = (jnp.transpose(weight.astype(jnp.float32), (2, 0, 3, 1))
           .reshape(3, Cout, 3 * Cin).astype(jnp.bfloat16))
    b2 = bias.astype(jnp.float32).reshape(Cout, 1)

    y, st = pl.pallas_call(
        _conv_stats_kernel,
        out_shape=(jax.ShapeDtypeStruct((N, Cout, HW), jnp.bfloat16),
                   jax.ShapeDtypeStruct((N, Cout, 2), jnp.float32)),
        name="conv3x3_stats",
        grid=(N,),
        in_specs=[pl.BlockSpec((1, Cin, H8, 8, W), lambda g: (g, 0, 0, 0, 0)),
                  pl.BlockSpec((3, Cout, 3 * Cin), lambda g: (0, 0, 0)),
                  pl.BlockSpec((Cout, 1), lambda g: (0, 0))],
        out_specs=(pl.BlockSpec((1, Cout, HW), lambda g: (g, 0, 0)),
                   pl.BlockSpec((1, Cout, 2), lambda g: (g, 0, 0))),
        scratch_shapes=[pltpu.VMEM((3 * Cin, HPW), jnp.bfloat16)],
        compiler_params=pltpu.CompilerParams(
            dimension_semantics=("parallel",),
            vmem_limit_bytes=64 * 1024 * 1024),
    )(xf, w_r, b2)

    cnt = float(N * HW)
    s = jnp.sum(st[:, :, 0], axis=0)
    ss = jnp.sum(st[:, :, 1], axis=0)
    mean = s / cnt
    var = jnp.maximum(ss / cnt - mean * mean, 0.0)
    scale = gamma.astype(jnp.float32) / jnp.sqrt(var + eps)
    shift = beta.astype(jnp.float32) - mean * scale
    scale2 = scale.reshape(Cout, 1)
    shift2 = shift.reshape(Cout, 1)

    CG = 2
    CB = Cout // CG
    out = pl.pallas_call(
        _bn_relu_kernel,
        out_shape=jax.ShapeDtypeStruct((N, Cout, H8, 8, W), jnp.float32),
        name="bn_relu",
        grid=(N, CG),
        in_specs=[pl.BlockSpec((1, CB, HW), lambda n, c: (n, c, 0)),
                  pl.BlockSpec((CB, 1), lambda n, c: (c, 0)),
                  pl.BlockSpec((CB, 1), lambda n, c: (c, 0))],
        out_specs=pl.BlockSpec((1, CB, H8, 8, W), lambda n, c: (n, c, 0, 0, 0)),
        compiler_params=pltpu.CompilerParams(
            dimension_semantics=("parallel", "parallel"),
            vmem_limit_bytes=64 * 1024 * 1024),
    )(y, scale2, shift2)

    return out.reshape(N, Cout, H, W)
